# baseline (speedup 1.0000x reference)
import functools

import jax
import jax.numpy as jnp
from jax import lax
from jax.experimental import pallas as pl
from jax.experimental.pallas import tpu as pltpu
from jax.experimental.pallas import tpu_sc as plsc

NUM_CORES = 2
NUM_SUBCORES = 16
NUM_WORKERS = NUM_CORES * NUM_SUBCORES
CHUNK = 128


@functools.partial(jax.jit, static_argnames=())
def _sc_gather(idx3, table):
    nw, n_chunks, chunk = idx3.shape
    d = table.shape[1]
    b_total = nw * n_chunks * chunk
    mesh = plsc.VectorSubcoreMesh(core_axis_name="c", subcore_axis_name="s")

    nbuf = 5
    lead = 3
    assert n_chunks % nbuf == 0

    @functools.partial(
        pl.kernel,
        out_type=jax.ShapeDtypeStruct((b_total, d), table.dtype),
        mesh=mesh,
        scratch_types=[
            pltpu.VMEM((n_chunks, chunk), jnp.int32),
            pltpu.VMEM((nbuf, chunk, d), table.dtype),
            [pltpu.SemaphoreType.DMA] * nbuf,
            [pltpu.SemaphoreType.DMA] * nbuf,
        ],
    )
    def body(idx_hbm, table_hbm, out_hbm, idx_v, rows_v, gsems, wsems):
        wid = lax.axis_index("s") * NUM_CORES + lax.axis_index("c")
        pltpu.sync_copy(idx_hbm.at[wid], idx_v)

        def gather(j, b):
            return pltpu.make_async_copy(
                table_hbm.at[idx_v.at[j]], rows_v.at[b], gsems[b]
            )

        def writeback(j, b):
            base = (wid * n_chunks + j) * chunk
            return pltpu.make_async_copy(
                rows_v.at[b], out_hbm.at[pl.ds(base, chunk)], wsems[b]
            )

        for j in range(lead):
            gather(j, j).start()

        def outer(i, carry):
            for b in range(nbuf):
                j = nbuf * i + b
                gather(j, b).wait()
                writeback(j, b).start()
                bn = (b + lead) % nbuf

                @pl.when(j + lead < n_chunks)
                def _():
                    @pl.when(j - (nbuf - lead) >= 0)
                    def _():
                        writeback(j - (nbuf - lead), bn).wait()

                    gather(j + lead, bn).start()
            return carry

        lax.fori_loop(0, n_chunks // nbuf, outer, 0)
        for j in range(n_chunks - nbuf, n_chunks):
            writeback(j, j % nbuf).wait()

    return body(idx3, table)


def kernel(token_ids, weight):
    b, s = token_ids.shape
    d = weight.shape[1]
    idx = token_ids.reshape(-1).astype(jnp.int32)
    idx3 = idx.reshape(NUM_WORKERS, -1, CHUNK)
    out = _sc_gather(idx3, weight)
    return out.reshape(b, s, d)


# speedup vs baseline: 1.6446x; 1.6446x over previous
import functools

import jax
import jax.numpy as jnp
from jax import lax
from jax.experimental import pallas as pl
from jax.experimental.pallas import tpu as pltpu
from jax.experimental.pallas import tpu_sc as plsc

NUM_CORES = 2
NUM_SUBCORES = 16
NUM_WORKERS = NUM_CORES * NUM_SUBCORES


@jax.jit
def _sc_gather(token_ids, table):
    bsz, seq = token_ids.shape
    d = table.shape[1]
    rows_per_w = bsz // NUM_WORKERS
    mesh = plsc.VectorSubcoreMesh(core_axis_name="c", subcore_axis_name="s")

    nbuf = 4
    lead = 2
    assert rows_per_w % nbuf == 0

    @functools.partial(
        pl.kernel,
        out_type=jax.ShapeDtypeStruct((bsz, seq, d), table.dtype),
        mesh=mesh,
        compiler_params=pltpu.CompilerParams(use_tc_tiling_on_sc=True),
        scratch_types=[
            pltpu.VMEM((rows_per_w, seq), jnp.int32),
            pltpu.VMEM((nbuf, seq, d), table.dtype),
            [pltpu.SemaphoreType.DMA] * nbuf,
            [pltpu.SemaphoreType.DMA] * nbuf,
        ],
    )
    def body(ids_hbm, table_hbm, out_hbm, idx_v, rows_v, gsems, wsems):
        wid = lax.axis_index("s") * NUM_CORES + lax.axis_index("c")
        base = wid * rows_per_w
        pltpu.sync_copy(ids_hbm.at[pl.ds(base, rows_per_w)], idx_v)

        def gather(j, b):
            return pltpu.make_async_copy(
                table_hbm.at[idx_v.at[j]], rows_v.at[b], gsems[b]
            )

        def writeback(j, b):
            return pltpu.make_async_copy(
                rows_v.at[b], out_hbm.at[base + j], wsems[b]
            )

        for j in range(lead):
            gather(j, j).start()

        def outer(i, carry):
            for b in range(nbuf):
                j = nbuf * i + b
                gather(j, b).wait()
                writeback(j, b).start()
                bn = (b + lead) % nbuf

                @pl.when(j + lead < rows_per_w)
                def _():
                    @pl.when(j - (nbuf - lead) >= 0)
                    def _():
                        writeback(j - (nbuf - lead), bn).wait()

                    gather(j + lead, bn).start()
            return carry

        lax.fori_loop(0, rows_per_w // nbuf, outer, 0)
        for j in range(rows_per_w - nbuf, rows_per_w):
            writeback(j, j % nbuf).wait()

    return body(token_ids, table)


def kernel(token_ids, weight):
    return _sc_gather(token_ids.astype(jnp.int32), weight)
